# baseline (speedup 1.0000x reference)
import jax
import jax.numpy as jnp
from jax.experimental import pallas as pl
from jax.experimental.pallas import tpu as pltpu


def _support_body(x_ref, w_ref, out_ref):
    out_ref[...] = jnp.dot(x_ref[...], w_ref[...],
                           preferred_element_type=jnp.float32)


def _spmm_body(adj_ref, s_ref, b_ref, out_ref):
    acc = jnp.dot(adj_ref[...], s_ref[...], preferred_element_type=jnp.float32)
    out_ref[...] = acc + b_ref[...]


def kernel(input, adj, weight, bias):
    n, din = input.shape
    dout = weight.shape[1]

    bm1 = 1000
    support = pl.pallas_call(
        _support_body,
        grid=(n // bm1,),
        in_specs=[
            pl.BlockSpec((bm1, din), lambda i: (i, 0)),
            pl.BlockSpec((din, dout), lambda i: (0, 0)),
        ],
        out_specs=pl.BlockSpec((bm1, dout), lambda i: (i, 0)),
        out_shape=jax.ShapeDtypeStruct((n, dout), jnp.float32),
        compiler_params=pltpu.CompilerParams(
            dimension_semantics=("parallel",)),
    )(input, weight)

    bm = 400
    out = pl.pallas_call(
        _spmm_body,
        grid=(n // bm,),
        in_specs=[
            pl.BlockSpec((bm, n), lambda i: (i, 0)),
            pl.BlockSpec((n, dout), lambda i: (0, 0)),
            pl.BlockSpec((1, dout), lambda i: (0, 0)),
        ],
        out_specs=pl.BlockSpec((bm, dout), lambda i: (i, 0)),
        out_shape=jax.ShapeDtypeStruct((n, dout), jnp.float32),
        compiler_params=pltpu.CompilerParams(
            dimension_semantics=("parallel",),
            vmem_limit_bytes=100 * 1024 * 1024),
    )(adj, support, bias.reshape(1, dout))
    return out


# speedup vs baseline: 1.0765x; 1.0765x over previous
import jax
import jax.numpy as jnp
from jax.experimental import pallas as pl
from jax.experimental.pallas import tpu as pltpu


def _fused_body(x_ref, w_ref, adj_ref, b_ref, out_ref, s_ref):
    @pl.when(pl.program_id(0) == 0)
    def _():
        s_ref[...] = jnp.dot(x_ref[...], w_ref[...],
                             preferred_element_type=jnp.float32)

    acc = jnp.dot(adj_ref[...], s_ref[...], preferred_element_type=jnp.float32)
    out_ref[...] = acc + b_ref[...]


def kernel(input, adj, weight, bias):
    n, din = input.shape
    dout = weight.shape[1]

    bm = 400
    out = pl.pallas_call(
        _fused_body,
        grid=(n // bm,),
        in_specs=[
            pl.BlockSpec((n, din), lambda i: (0, 0)),
            pl.BlockSpec((din, dout), lambda i: (0, 0)),
            pl.BlockSpec((bm, n), lambda i: (i, 0)),
            pl.BlockSpec((1, dout), lambda i: (0, 0)),
        ],
        out_specs=pl.BlockSpec((bm, dout), lambda i: (i, 0)),
        out_shape=jax.ShapeDtypeStruct((n, dout), jnp.float32),
        scratch_shapes=[pltpu.VMEM((n, dout), jnp.float32)],
        compiler_params=pltpu.CompilerParams(
            dimension_semantics=("arbitrary",),
            vmem_limit_bytes=100 * 1024 * 1024),
    )(input, weight, adj, bias.reshape(1, dout))
    return out
